# baseline (device time: 443450 ns/iter reference)
import jax
import jax.numpy as jnp
from jax import lax
from jax.experimental import pallas as pl
from jax.experimental.pallas import tpu as pltpu

N_DEV = 4
SEQ = 2048
D = 1024
HQ = 8
DH = 128
KV_UNIT = 1024
SCALE = 0.08838834764831843
_FREQ_C = -2.0 * 9.210340371976184 / DH


def _rope_tables(offs_f32):
    pos = (lax.broadcasted_iota(jnp.int32, (SEQ, DH), 0).astype(jnp.float32)
           + offs_f32)
    d = lax.broadcasted_iota(jnp.int32, (SEQ, DH), 1)
    pair = (d // 2).astype(jnp.float32)
    ang = pos * jnp.exp(pair * _FREQ_C)
    return jnp.cos(ang), jnp.sin(ang)


def _proj(x, w):
    return lax.dot_general(x, w, (((1,), (0,)), ((), ())),
                           preferred_element_type=jnp.float32)


def _body(x_hbm, wq, wkv, wo, out_ref, xg,
          xc, q_ref, acc, copy_sem, send_sems, recv_sems):
    my = lax.axis_index("i")
    right = jnp.mod(my + 1, N_DEV)
    left = jnp.mod(my - 1, N_DEV)

    cp = pltpu.make_async_copy(x_hbm.at[0], xc, copy_sem)
    cp.start()
    cp.wait()

    barrier = pltpu.get_barrier_semaphore()
    for nbr in (left, right):
        pl.semaphore_signal(barrier, inc=1, device_id=(nbr,),
                            device_id_type=pl.DeviceIdType.MESH)
    pl.semaphore_wait(barrier, 2)

    x0 = xc[...]
    cos0, sin0 = _rope_tables((my * SEQ).astype(jnp.float32))

    def init_head(h, _):
        qq = _proj(x0, wq[h])
        q_ref[h] = ((qq[:, :DH] * cos0 + qq[:, DH:] * sin0)
                    * SCALE).astype(jnp.bfloat16)
        acc[h] = jnp.zeros((SEQ, 2 * DH), jnp.float32)
        return 0

    lax.fori_loop(0, HQ, init_head, 0)

    ones = jnp.ones((KV_UNIT, DH), jnp.bfloat16)

    def flash_chunk(origin):
        offs = (origin * SEQ).astype(jnp.float32)
        cos, sin = _rope_tables(offs)
        xck = xc[...]

        def head_body(h, _):
            kv = _proj(xck, wkv[h])
            kb = (kv[:, :DH] * cos + kv[:, DH:2 * DH] * sin).astype(jnp.bfloat16)
            vb = kv[:, 2 * DH:].astype(jnp.bfloat16)
            qh = q_ref[h]
            contrib = None
            for j in range(SEQ // KV_UNIT):
                r0 = j * KV_UNIT
                vaug = jnp.concatenate([vb[r0:r0 + KV_UNIT, :], ones], axis=1)
                s_ = lax.dot_general(qh, kb[r0:r0 + KV_UNIT, :],
                                     (((1,), (1,)), ((), ())),
                                     preferred_element_type=jnp.float32)
                p = jnp.exp(s_).astype(jnp.bfloat16)
                d_ = lax.dot_general(p, vaug, (((1,), (0,)), ((), ())),
                                     preferred_element_type=jnp.float32)
                contrib = d_ if contrib is None else contrib + d_
            acc[h] = acc[h] + contrib
            return 0

        lax.fori_loop(0, HQ, head_body, 0)

    def hop_rdma(hop):
        return pltpu.make_async_remote_copy(
            src_ref=x_hbm.at[0] if hop == 0 else xg.at[hop - 1],
            dst_ref=xg.at[hop],
            send_sem=send_sems.at[hop],
            recv_sem=recv_sems.at[hop],
            device_id=(right,),
            device_id_type=pl.DeviceIdType.MESH,
        )

    rdmas = [hop_rdma(0)]
    rdmas[0].start()
    flash_chunk(my)

    for hop in range(N_DEV - 1):
        rdmas[hop].wait_recv()
        if hop + 1 < N_DEV - 1:
            nxt = hop_rdma(hop + 1)
            nxt.start()
            rdmas.append(nxt)
        cp = pltpu.make_async_copy(xg.at[hop], xc, copy_sem)
        cp.start()
        cp.wait()
        flash_chunk(jnp.mod(my - hop - 1, N_DEV))

    for r in rdmas:
        r.wait_send()

    xc[...] = jnp.zeros((SEQ, D), jnp.bfloat16)

    def out_head(h, _):
        a = acc[h]
        ctx = (a[:, :DH] / a[:, DH:DH + 1]).astype(jnp.bfloat16)
        xc[...] = xc[...] + lax.dot_general(
            ctx, wo[h], (((1,), (0,)), ((), ())),
            preferred_element_type=jnp.float32).astype(jnp.bfloat16)
        return 0

    lax.fori_loop(0, HQ, out_head, 0)

    cp = pltpu.make_async_copy(xc, out_ref.at[0], copy_sem)
    cp.start()
    cp.wait()


def kernel(x, Wq, Wk, Wv, Wo):
    xb = x.astype(jnp.bfloat16)
    def per_head(w):
        return w.astype(jnp.bfloat16).reshape(D, HQ, DH).transpose(1, 0, 2)

    def rot(w):
        wr = w.reshape(D, HQ, DH // 2, 2)
        return (jnp.stack([-wr[..., 1], wr[..., 0]], axis=-1)
                .reshape(D, HQ, DH).astype(jnp.bfloat16).transpose(1, 0, 2))

    wq = jnp.concatenate([per_head(Wq), rot(Wq)], axis=2)
    wkv = jnp.concatenate([per_head(Wk), rot(Wk), per_head(Wv)],
                          axis=2)
    wo = Wo.astype(jnp.bfloat16).reshape(HQ, DH, D)
    out, _ = pl.pallas_call(
        _body,
        out_shape=[
            jax.ShapeDtypeStruct((1, SEQ, D), jnp.bfloat16),
            jax.ShapeDtypeStruct((N_DEV - 1, SEQ, D), jnp.bfloat16),
        ],
        in_specs=[
            pl.BlockSpec(memory_space=pltpu.MemorySpace.HBM),
            pl.BlockSpec(memory_space=pltpu.MemorySpace.VMEM),
            pl.BlockSpec(memory_space=pltpu.MemorySpace.VMEM),
            pl.BlockSpec(memory_space=pltpu.MemorySpace.VMEM),
        ],
        out_specs=[
            pl.BlockSpec(memory_space=pltpu.MemorySpace.HBM),
            pl.BlockSpec(memory_space=pltpu.MemorySpace.HBM),
        ],
        scratch_shapes=[
            pltpu.VMEM((SEQ, D), jnp.bfloat16),
            pltpu.VMEM((HQ, SEQ, DH), jnp.bfloat16),
            pltpu.VMEM((HQ, SEQ, 2 * DH), jnp.float32),
            pltpu.SemaphoreType.DMA,
            pltpu.SemaphoreType.DMA((N_DEV - 1,)),
            pltpu.SemaphoreType.DMA((N_DEV - 1,)),
        ],
        compiler_params=pltpu.CompilerParams(
            collective_id=0, vmem_limit_bytes=100 * 1024 * 1024),
    )(xb, wq, wkv, wo)
    return out


# device time: 269699 ns/iter; 1.6442x vs baseline; 1.6442x over previous
import jax
import jax.numpy as jnp
from jax import lax
from jax.experimental import pallas as pl
from jax.experimental.pallas import tpu as pltpu

N_DEV = 4
SEQ = 2048
D = 1024
HQ = 8
DH = 128
KV_UNIT = 512
SCALE = 0.08838834764831843
_FREQ_C = -2.0 * 9.210340371976184 / DH


def _rope_tables(offs_f32):
    pos = (lax.broadcasted_iota(jnp.int32, (SEQ, DH), 0).astype(jnp.float32)
           + offs_f32)
    d = lax.broadcasted_iota(jnp.int32, (SEQ, DH), 1)
    pair = (d // 2).astype(jnp.float32)
    ang = pos * jnp.exp(pair * _FREQ_C)
    even = (d % 2) == 0
    return jnp.cos(ang), jnp.sin(ang), even


def _rope(t, cos, sin, even):
    rot = jnp.where(even, -jnp.roll(t, -1, axis=1), jnp.roll(t, 1, axis=1))
    return t * cos + rot * sin


def _proj(x, w):
    return lax.dot_general(x, w, (((1,), (0,)), ((), ())),
                           preferred_element_type=jnp.float32)


def _body(x_hbm, wq, wkv, wo, out_ref, xg,
          xc, q_ref, acc, copy_sem, send_sems, recv_sems):
    my = lax.axis_index("i")
    right = jnp.mod(my + 1, N_DEV)
    left = jnp.mod(my - 1, N_DEV)

    cp = pltpu.make_async_copy(x_hbm.at[0], xc, copy_sem)
    cp.start()
    cp.wait()

    barrier = pltpu.get_barrier_semaphore()
    for nbr in (left, right):
        pl.semaphore_signal(barrier, inc=1, device_id=(nbr,),
                            device_id_type=pl.DeviceIdType.MESH)
    pl.semaphore_wait(barrier, 2)

    x0 = xc[...]
    cos0, sin0, even0 = _rope_tables((my * SEQ).astype(jnp.float32))

    def init_head(h, _):
        qp = _proj(x0, wq[h])
        q_ref[h] = (_rope(qp, cos0, sin0, even0) * SCALE).astype(jnp.bfloat16)
        acc[h] = jnp.zeros((SEQ, 2 * DH), jnp.float32)
        return 0

    lax.fori_loop(0, HQ, init_head, 0)

    ones = jnp.ones((KV_UNIT, DH), jnp.bfloat16)

    def flash_chunk(origin):
        offs = (origin * SEQ).astype(jnp.float32)
        cos, sin, even = _rope_tables(offs)
        xck = xc[...]

        def head_pair_body(i, _):
            h0 = 2 * i
            h1 = h0 + 1
            kv0 = _proj(xck, wkv[h0])
            kv1 = _proj(xck, wkv[h1])
            kb0 = _rope(kv0[:, :DH], cos, sin, even).astype(jnp.bfloat16)
            vb0 = kv0[:, DH:].astype(jnp.bfloat16)
            kb1 = _rope(kv1[:, :DH], cos, sin, even).astype(jnp.bfloat16)
            vb1 = kv1[:, DH:].astype(jnp.bfloat16)
            q0 = q_ref[h0]
            q1 = q_ref[h1]
            c0 = c1 = None
            for j in range(SEQ // KV_UNIT):
                r0 = j * KV_UNIT
                va0 = jnp.concatenate([vb0[r0:r0 + KV_UNIT, :], ones], axis=1)
                va1 = jnp.concatenate([vb1[r0:r0 + KV_UNIT, :], ones], axis=1)
                s0 = lax.dot_general(q0, kb0[r0:r0 + KV_UNIT, :],
                                     (((1,), (1,)), ((), ())),
                                     preferred_element_type=jnp.float32)
                s1 = lax.dot_general(q1, kb1[r0:r0 + KV_UNIT, :],
                                     (((1,), (1,)), ((), ())),
                                     preferred_element_type=jnp.float32)
                p0 = jnp.exp(s0).astype(jnp.bfloat16)
                p1 = jnp.exp(s1).astype(jnp.bfloat16)
                d0 = lax.dot_general(p0, va0, (((1,), (0,)), ((), ())),
                                     preferred_element_type=jnp.float32)
                d1 = lax.dot_general(p1, va1, (((1,), (0,)), ((), ())),
                                     preferred_element_type=jnp.float32)
                c0 = d0 if c0 is None else c0 + d0
                c1 = d1 if c1 is None else c1 + d1
            acc[h0] = acc[h0] + c0
            acc[h1] = acc[h1] + c1
            return 0

        lax.fori_loop(0, HQ // 2, head_pair_body, 0)

    def hop_rdma(hop):
        return pltpu.make_async_remote_copy(
            src_ref=x_hbm.at[0] if hop == 0 else xg.at[hop - 1],
            dst_ref=xg.at[hop],
            send_sem=send_sems.at[hop],
            recv_sem=recv_sems.at[hop],
            device_id=(right,),
            device_id_type=pl.DeviceIdType.MESH,
        )

    rdmas = [hop_rdma(0)]
    rdmas[0].start()
    flash_chunk(my)

    for hop in range(N_DEV - 1):
        rdmas[hop].wait_recv()
        if hop + 1 < N_DEV - 1:
            nxt = hop_rdma(hop + 1)
            nxt.start()
            rdmas.append(nxt)
        cp = pltpu.make_async_copy(xg.at[hop], xc, copy_sem)
        cp.start()
        cp.wait()
        flash_chunk(jnp.mod(my - hop - 1, N_DEV))

    for r in rdmas:
        r.wait_send()

    xc[...] = jnp.zeros((SEQ, D), jnp.bfloat16)

    def out_head(h, _):
        a = acc[h]
        ctx = (a[:, :DH] / a[:, DH:DH + 1]).astype(jnp.bfloat16)
        xc[...] = xc[...] + lax.dot_general(
            ctx, wo[h], (((1,), (0,)), ((), ())),
            preferred_element_type=jnp.float32).astype(jnp.bfloat16)
        return 0

    lax.fori_loop(0, HQ, out_head, 0)

    cp = pltpu.make_async_copy(xc, out_ref.at[0], copy_sem)
    cp.start()
    cp.wait()


def kernel(x, Wq, Wk, Wv, Wo):
    xb = x.astype(jnp.bfloat16)
    def per_head(w):
        return w.astype(jnp.bfloat16).reshape(D, HQ, DH).transpose(1, 0, 2)

    wq = per_head(Wq)
    wkv = jnp.concatenate([per_head(Wk), per_head(Wv)], axis=2)
    wo = Wo.astype(jnp.bfloat16).reshape(HQ, DH, D)
    out, _ = pl.pallas_call(
        _body,
        out_shape=[
            jax.ShapeDtypeStruct((1, SEQ, D), jnp.bfloat16),
            jax.ShapeDtypeStruct((N_DEV - 1, SEQ, D), jnp.bfloat16),
        ],
        in_specs=[
            pl.BlockSpec(memory_space=pltpu.MemorySpace.HBM),
            pl.BlockSpec(memory_space=pltpu.MemorySpace.VMEM),
            pl.BlockSpec(memory_space=pltpu.MemorySpace.VMEM),
            pl.BlockSpec(memory_space=pltpu.MemorySpace.VMEM),
        ],
        out_specs=[
            pl.BlockSpec(memory_space=pltpu.MemorySpace.HBM),
            pl.BlockSpec(memory_space=pltpu.MemorySpace.HBM),
        ],
        scratch_shapes=[
            pltpu.VMEM((SEQ, D), jnp.bfloat16),
            pltpu.VMEM((HQ, SEQ, DH), jnp.bfloat16),
            pltpu.VMEM((HQ, SEQ, 2 * DH), jnp.float32),
            pltpu.SemaphoreType.DMA,
            pltpu.SemaphoreType.DMA((N_DEV - 1,)),
            pltpu.SemaphoreType.DMA((N_DEV - 1,)),
        ],
        compiler_params=pltpu.CompilerParams(
            collective_id=0, vmem_limit_bytes=100 * 1024 * 1024),
    )(xb, wq, wkv, wo)
    return out
